# baseline (speedup 1.0000x reference)
import jax
import jax.numpy as jnp
from jax import lax
from jax.experimental import pallas as pl
from jax.experimental.pallas import tpu as pltpu
from jax.experimental.pallas import tpu_sc as plsc

N = 10000
F = 128
FH = 64
E = 320000
KSTEPS = 10
ALPHA = 0.1
BN_EPS = 1e-5

NC = 2
NT = 16
NP = 10240
EPT = E // NT
CH = 64
NCHUNK = (EPT + CH - 1) // CH
EPAD = NCHUNK * CH - EPT
ROWS_PT = NP // NT
RCH = 32
NRCH = ROWS_PT // RCH
DWIN = ROWS_PT
LBUF = DWIN + 16


def _rsqrt16(d):
    i = lax.bitcast_convert_type(d, jnp.int32)
    y = lax.bitcast_convert_type(jnp.int32(0x5F3759DF) - (i >> 1), jnp.float32)
    for _ in range(3):
        y = y * (1.5 - 0.5 * d * y * y)
    return y


def _sc_propagate(srcp, dstp, h0):
    mesh = plsc.VectorSubcoreMesh(
        core_axis_name="c", subcore_axis_name="s", num_cores=NC,
        num_subcores=NT)

    def body(srcp_hbm, dstp_hbm, h0_hbm, g_hbm, inv_hbm,
             acc_sp, odeg_sp, ideg_sp,
             src_idx, dst_idx, gbuf0, gbuf1, gbuf2, gbuf3, gbuf4, gbuf5,
             accb0, accb1, h0b0, h0b1, gnb0, gnb1, zbuf,
             zflat, ones, dbuf, ns_l, nd_l, inv_l,
             sem0, sem1, sem2, sem3, sem4, sem5, sra0, sra1, srh0, srh1, swg0, swg1):
        c = lax.axis_index("c")
        t = lax.axis_index("s")
        base = pl.multiple_of(t * ROWS_PT, RCH)

        pltpu.sync_copy(srcp_hbm.at[t], src_idx)
        pltpu.sync_copy(dstp_hbm.at[t], dst_idx)

        zv = jnp.zeros((16,), jnp.float32)

        def zrow(i, _):
            for v in range(4):
                zbuf[i, pl.ds(16 * v, 16)] = zv
            return 0
        lax.fori_loop(0, RCH, zrow, 0)

        def onesf(i, _):
            ones[pl.ds(pl.multiple_of(i * 16, 16), 16)] = zv + 1.0
            return 0
        lax.fori_loop(0, CH // 16, onesf, 0)

        def zfl(i, _):
            zflat[pl.ds(pl.multiple_of(i * 16, 16), 16)] = zv
            return 0
        lax.fori_loop(0, RCH // 16, zfl, 0)

        def zsp(u, _):
            rb = pl.multiple_of(base + u * RCH, RCH)
            pltpu.sync_copy(zbuf, acc_sp.at[pl.ds(rb, RCH), :])
            pltpu.sync_copy(zflat, odeg_sp.at[pl.ds(rb, RCH)])
            pltpu.sync_copy(zflat, ideg_sp.at[pl.ds(rb, RCH)])
            return 0
        lax.fori_loop(0, NRCH, zsp, 0)
        plsc.subcore_barrier()

        def hissue(j, so, si):
            pltpu.async_copy(ones, odeg_sp.at[src_idx.at[j]], so, add=True)
            pltpu.async_copy(ones, ideg_sp.at[dst_idx.at[j]], si, add=True)

        def hdrain(j, so, si):
            pltpu.make_async_copy(ones, odeg_sp.at[src_idx.at[j]], so).wait()
            pltpu.make_async_copy(ones, ideg_sp.at[dst_idx.at[j]], si).wait()

        hissue(0, sra0, srh0)

        def hist_chunk(j, _):
            nxt = j + 1

            @pl.when(jnp.logical_and(nxt < NCHUNK, nxt % 2 == 1))
            def _():
                hissue(nxt, sra1, srh1)

            @pl.when(jnp.logical_and(nxt < NCHUNK, nxt % 2 == 0))
            def _():
                hissue(nxt, sra0, srh0)

            @pl.when(j % 2 == 0)
            def _():
                hdrain(j, sra0, srh0)

            @pl.when(j % 2 == 1)
            def _():
                hdrain(j, sra1, srh1)
            return 0
        lax.fori_loop(0, NCHUNK, hist_chunk, 0)
        plsc.subcore_barrier()

        pltpu.sync_copy(odeg_sp.at[pl.ds(base, DWIN)], dbuf)

        def nsv(i, _):
            o = pl.multiple_of(i * 16, 16)
            d = jnp.maximum(dbuf[pl.ds(o, 16)], 1.0)
            y = _rsqrt16(d)
            ns_l[pl.ds(o, 16)] = y
            inv_l[pl.ds(o, 16)] = d * y
            return 0
        lax.fori_loop(0, DWIN // 16, nsv, 0)

        pltpu.sync_copy(ideg_sp.at[pl.ds(base, DWIN)], dbuf)

        def ndv(i, _):
            o = pl.multiple_of(i * 16, 16)
            d = jnp.maximum(dbuf[pl.ds(o, 16)], 1.0)
            nd_l[pl.ds(o, 16)] = _rsqrt16(d)
            return 0
        lax.fori_loop(0, DWIN // 16, ndv, 0)

        def g0chunk(u, _):
            rb = pl.multiple_of(base + u * RCH, RCH)
            pltpu.sync_copy(h0_hbm.at[c].at[pl.ds(rb, RCH), :], h0b0)

            def g0row(r, _):
                s = ns_l[pl.ds(u * RCH + r, 16)][0]
                for v in range(4):
                    h0v = h0b0[r, pl.ds(16 * v, 16)]
                    gnb0[r, pl.ds(16 * v, 16)] = s * h0v
                return 0
            lax.fori_loop(0, RCH, g0row, 0)
            pltpu.sync_copy(gnb0, g_hbm.at[c].at[pl.ds(rb, RCH), :])
            return 0
        lax.fori_loop(0, NRCH, g0chunk, 0)
        plsc.subcore_barrier()

        def issue(j, buf, sem):
            return pltpu.async_copy(g_hbm.at[c].at[src_idx.at[j]], buf, sem)

        def bread(u, ab, hb, sa, sh):
            rb = pl.multiple_of(base + u * RCH, RCH)
            pltpu.async_copy(acc_sp.at[pl.ds(rb, RCH), :], ab, sa)
            pltpu.async_copy(h0_hbm.at[c].at[pl.ds(rb, RCH), :], hb, sh)

        def bwait(u, ab, hb, sa, sh):
            rb = pl.multiple_of(base + u * RCH, RCH)
            pltpu.make_async_copy(
                acc_sp.at[pl.ds(rb, RCH), :], ab, sa).wait()
            pltpu.make_async_copy(
                h0_hbm.at[c].at[pl.ds(rb, RCH), :], hb, sh).wait()

        def gwrite(u, gb, sw):
            rb = pl.multiple_of(base + u * RCH, RCH)
            return pltpu.async_copy(gb, g_hbm.at[c].at[pl.ds(rb, RCH), :], sw)

        def gwdrain(u, gb, sw):
            rb = pl.multiple_of(base + u * RCH, RCH)
            pltpu.make_async_copy(gb, g_hbm.at[c].at[pl.ds(rb, RCH), :],
                                  sw).wait()

        gbufs = (gbuf0, gbuf1, gbuf2, gbuf3, gbuf4, gbuf5)
        gsems = (sem0, sem1, sem2, sem3, sem4, sem5)

        def step(_k, _):
            for bb in range(5):
                issue(bb, gbufs[bb], gsems[bb])

            def chunk(j, _):
                nxt = j + 5
                for b in range(6):
                    @pl.when(jnp.logical_and(nxt < NCHUNK, nxt % 6 == b))
                    def _(b=b):
                        issue(nxt, gbufs[b], gsems[b])

                for b in range(6):
                    @pl.when(j % 6 == b)
                    def _(b=b):
                        pltpu.make_async_copy(
                            g_hbm.at[c].at[src_idx.at[j]],
                            gbufs[b], gsems[b]).wait()
                        pltpu.sync_copy(gbufs[b], acc_sp.at[dst_idx.at[j]],
                                        add=True)
                return 0
            lax.fori_loop(0, NCHUNK, chunk, 0)
            plsc.subcore_barrier()

            bread(0, accb0, h0b0, sra0, srh0)

            def bcompute(u, ab, hb, gb):
                def brow(r, _):
                    li = u * RCH + r
                    s_ns = ns_l[pl.ds(li, 16)][0]
                    c1 = (1.0 - ALPHA) * s_ns * nd_l[pl.ds(li, 16)][0]
                    c2 = ALPHA * s_ns
                    for v in range(4):
                        av = ab[r, pl.ds(16 * v, 16)]
                        hv = hb[r, pl.ds(16 * v, 16)]
                        gb[r, pl.ds(16 * v, 16)] = c1 * av + c2 * hv
                    return 0
                lax.fori_loop(0, RCH, brow, 0)

            def bchunk(u, _):
                nxt = u + 1

                @pl.when(jnp.logical_and(nxt < NRCH, nxt % 2 == 1))
                def _():
                    bread(nxt, accb1, h0b1, sra1, srh1)

                @pl.when(jnp.logical_and(nxt < NRCH, nxt % 2 == 0))
                def _():
                    bread(nxt, accb0, h0b0, sra0, srh0)

                @pl.when(u % 2 == 0)
                def _():
                    bwait(u, accb0, h0b0, sra0, srh0)

                    @pl.when(u >= 2)
                    def _():
                        gwdrain(u - 2, gnb0, swg0)
                    bcompute(u, accb0, h0b0, gnb0)
                    gwrite(u, gnb0, swg0)

                @pl.when(u % 2 == 1)
                def _():
                    bwait(u, accb1, h0b1, sra1, srh1)

                    @pl.when(u >= 2)
                    def _():
                        gwdrain(u - 2, gnb1, swg1)
                    bcompute(u, accb1, h0b1, gnb1)
                    gwrite(u, gnb1, swg1)

                rb = pl.multiple_of(base + u * RCH, RCH)
                pltpu.sync_copy(zbuf, acc_sp.at[pl.ds(rb, RCH), :])
                return 0
            lax.fori_loop(0, NRCH, bchunk, 0)
            gwdrain(NRCH - 2, gnb0, swg0)
            gwdrain(NRCH - 1, gnb1, swg1)
            plsc.subcore_barrier()
            return 0
        lax.fori_loop(0, KSTEPS, step, 0)

        @pl.when(c == 0)
        def _():
            pltpu.sync_copy(inv_l.at[pl.ds(0, DWIN)],
                            inv_hbm.at[pl.ds(base, DWIN)])

    f32 = jnp.float32
    kern = pl.kernel(
        body,
        out_type=[
            jax.ShapeDtypeStruct((NC, NP, FH), f32),
            jax.ShapeDtypeStruct((NP,), f32),
        ],
        mesh=mesh,
        compiler_params=pltpu.CompilerParams(use_tc_tiling_on_sc=False),
        scratch_types=[
            pltpu.VMEM_SHARED((NP, FH), f32),
            pltpu.VMEM_SHARED((NP,), f32),
            pltpu.VMEM_SHARED((NP,), f32),
            pltpu.VMEM((NCHUNK, CH), jnp.int32),
            pltpu.VMEM((NCHUNK, CH), jnp.int32),
            pltpu.VMEM((CH, FH), f32),
            pltpu.VMEM((CH, FH), f32),
            pltpu.VMEM((CH, FH), f32),
            pltpu.VMEM((CH, FH), f32),
            pltpu.VMEM((CH, FH), f32),
            pltpu.VMEM((CH, FH), f32),
            pltpu.VMEM((RCH, FH), f32),
            pltpu.VMEM((RCH, FH), f32),
            pltpu.VMEM((RCH, FH), f32),
            pltpu.VMEM((RCH, FH), f32),
            pltpu.VMEM((RCH, FH), f32),
            pltpu.VMEM((RCH, FH), f32),
            pltpu.VMEM((RCH, FH), f32),
            pltpu.VMEM((RCH,), f32),
            pltpu.VMEM((CH,), f32),
            pltpu.VMEM((DWIN,), f32),
            pltpu.VMEM((LBUF,), f32),
            pltpu.VMEM((LBUF,), f32),
            pltpu.VMEM((LBUF,), f32),
            pltpu.SemaphoreType.DMA,
            pltpu.SemaphoreType.DMA,
            pltpu.SemaphoreType.DMA,
            pltpu.SemaphoreType.DMA,
            pltpu.SemaphoreType.DMA,
            pltpu.SemaphoreType.DMA,
            pltpu.SemaphoreType.DMA,
            pltpu.SemaphoreType.DMA,
            pltpu.SemaphoreType.DMA,
            pltpu.SemaphoreType.DMA,
            pltpu.SemaphoreType.DMA,
            pltpu.SemaphoreType.DMA,
        ],
    )
    g, inv_ns = kern(srcp, dstp, h0)
    return g, inv_ns


def _tc_linear(feat, W, b):
    def body(feat_ref, w_ref, b_ref, out_ref):
        h = lax.dot_general(feat_ref[...], w_ref[...],
                            (((1,), (1,)), ((), ())),
                            preferred_element_type=jnp.float32)
        h = jnp.maximum(h + b_ref[...][None, :], 0.0)
        out_ref[0, :N] = h[:, :FH]
        out_ref[1, :N] = h[:, FH:]
        pad = jnp.zeros((NP - N, FH), jnp.float32)
        out_ref[0, N:] = pad
        out_ref[1, N:] = pad

    return pl.pallas_call(
        body,
        out_shape=jax.ShapeDtypeStruct((NC, NP, FH), jnp.float32),
    )(feat, W, b)


def _tc_batchnorm(hfin, inv_ns, gamma, beta):
    def body(h_ref, inv_ref, g_ref, b_ref, out_ref):
        inv = inv_ref[pl.ds(0, N)][:, None]
        for half in range(NC):
            x = h_ref[half, :N] * inv
            m = jnp.mean(x, axis=0)
            var = jnp.mean((x - m[None, :]) ** 2, axis=0)
            scale = lax.rsqrt(var + BN_EPS) * g_ref[pl.ds(half * FH, FH)]
            out_ref[:, pl.ds(half * FH, FH)] = (
                (x - m[None, :]) * scale[None, :]
                + b_ref[pl.ds(half * FH, FH)][None, :])

    return pl.pallas_call(
        body,
        out_shape=jax.ShapeDtypeStruct((N, F), jnp.float32),
    )(hfin, inv_ns, gamma, beta)


@jax.jit
def kernel(feat, edge_index, W, b, gamma, beta):
    src = edge_index[0].astype(jnp.int32)
    dst = edge_index[1].astype(jnp.int32)
    pad = (N + jnp.arange(NT, dtype=jnp.int32))[:, None] * jnp.ones(
        (1, EPAD), jnp.int32)
    srcp = jnp.concatenate([src.reshape(NT, EPT), pad], axis=1)
    srcp = srcp.reshape(NT, NCHUNK, CH)
    dstp = jnp.concatenate([dst.reshape(NT, EPT), pad], axis=1)
    dstp = dstp.reshape(NT, NCHUNK, CH)

    h0 = _tc_linear(feat, W, b)
    g, inv_ns = _sc_propagate(srcp, dstp, h0)
    return _tc_batchnorm(g, inv_ns, gamma, beta)


# speedup vs baseline: 1.0103x; 1.0103x over previous
import jax
import jax.numpy as jnp
from jax import lax
from jax.experimental import pallas as pl
from jax.experimental.pallas import tpu as pltpu
from jax.experimental.pallas import tpu_sc as plsc

N = 10000
F = 128
FH = 64
E = 320000
KSTEPS = 10
ALPHA = 0.1
BN_EPS = 1e-5

NC = 2
NT = 16
NP = 10240
EPT = E // NT
CH = 96
NCHUNK = (EPT + CH - 1) // CH
EPAD = NCHUNK * CH - EPT
ROWS_PT = NP // NT
RCH = 32
NRCH = ROWS_PT // RCH
DWIN = ROWS_PT
LBUF = DWIN + 16


def _rsqrt16(d):
    i = lax.bitcast_convert_type(d, jnp.int32)
    y = lax.bitcast_convert_type(jnp.int32(0x5F3759DF) - (i >> 1), jnp.float32)
    for _ in range(3):
        y = y * (1.5 - 0.5 * d * y * y)
    return y


def _sc_propagate(srcp, dstp, h0):
    mesh = plsc.VectorSubcoreMesh(
        core_axis_name="c", subcore_axis_name="s", num_cores=NC,
        num_subcores=NT)

    def body(srcp_hbm, dstp_hbm, h0_hbm, g_hbm, inv_hbm,
             acc_sp, odeg_sp, ideg_sp,
             src_idx, dst_idx, gbuf0, gbuf1, gbuf2, gbuf3, gbuf4,
             accb0, accb1, h0b0, h0b1, gnb0, gnb1, zbuf,
             zflat, ones, dbuf, ns_l, nd_l, inv_l,
             sem0, sem1, sem2, sem3, sem4, sra0, sra1, srh0, srh1, swg0, swg1):
        c = lax.axis_index("c")
        t = lax.axis_index("s")
        base = pl.multiple_of(t * ROWS_PT, RCH)

        pltpu.sync_copy(srcp_hbm.at[t], src_idx)
        pltpu.sync_copy(dstp_hbm.at[t], dst_idx)

        zv = jnp.zeros((16,), jnp.float32)

        def zrow(i, _):
            for v in range(4):
                zbuf[i, pl.ds(16 * v, 16)] = zv
            return 0
        lax.fori_loop(0, RCH, zrow, 0)

        def onesf(i, _):
            ones[pl.ds(pl.multiple_of(i * 16, 16), 16)] = zv + 1.0
            return 0
        lax.fori_loop(0, CH // 16, onesf, 0)

        def zfl(i, _):
            zflat[pl.ds(pl.multiple_of(i * 16, 16), 16)] = zv
            return 0
        lax.fori_loop(0, RCH // 16, zfl, 0)

        def zsp(u, _):
            rb = pl.multiple_of(base + u * RCH, RCH)
            pltpu.sync_copy(zbuf, acc_sp.at[pl.ds(rb, RCH), :])
            pltpu.sync_copy(zflat, odeg_sp.at[pl.ds(rb, RCH)])
            pltpu.sync_copy(zflat, ideg_sp.at[pl.ds(rb, RCH)])
            return 0
        lax.fori_loop(0, NRCH, zsp, 0)
        plsc.subcore_barrier()

        def hissue(j, so, si):
            pltpu.async_copy(ones, odeg_sp.at[src_idx.at[j]], so, add=True)
            pltpu.async_copy(ones, ideg_sp.at[dst_idx.at[j]], si, add=True)

        def hdrain(j, so, si):
            pltpu.make_async_copy(ones, odeg_sp.at[src_idx.at[j]], so).wait()
            pltpu.make_async_copy(ones, ideg_sp.at[dst_idx.at[j]], si).wait()

        hissue(0, sra0, srh0)

        def hist_chunk(j, _):
            nxt = j + 1

            @pl.when(jnp.logical_and(nxt < NCHUNK, nxt % 2 == 1))
            def _():
                hissue(nxt, sra1, srh1)

            @pl.when(jnp.logical_and(nxt < NCHUNK, nxt % 2 == 0))
            def _():
                hissue(nxt, sra0, srh0)

            @pl.when(j % 2 == 0)
            def _():
                hdrain(j, sra0, srh0)

            @pl.when(j % 2 == 1)
            def _():
                hdrain(j, sra1, srh1)
            return 0
        lax.fori_loop(0, NCHUNK, hist_chunk, 0)
        plsc.subcore_barrier()

        pltpu.sync_copy(odeg_sp.at[pl.ds(base, DWIN)], dbuf)

        def nsv(i, _):
            o = pl.multiple_of(i * 16, 16)
            d = jnp.maximum(dbuf[pl.ds(o, 16)], 1.0)
            y = _rsqrt16(d)
            ns_l[pl.ds(o, 16)] = y
            inv_l[pl.ds(o, 16)] = d * y
            return 0
        lax.fori_loop(0, DWIN // 16, nsv, 0)

        pltpu.sync_copy(ideg_sp.at[pl.ds(base, DWIN)], dbuf)

        def ndv(i, _):
            o = pl.multiple_of(i * 16, 16)
            d = jnp.maximum(dbuf[pl.ds(o, 16)], 1.0)
            nd_l[pl.ds(o, 16)] = _rsqrt16(d)
            return 0
        lax.fori_loop(0, DWIN // 16, ndv, 0)

        def g0chunk(u, _):
            rb = pl.multiple_of(base + u * RCH, RCH)
            pltpu.sync_copy(h0_hbm.at[c].at[pl.ds(rb, RCH), :], h0b0)

            def g0row(r, _):
                s = ns_l[pl.ds(u * RCH + r, 16)][0]
                for v in range(4):
                    h0v = h0b0[r, pl.ds(16 * v, 16)]
                    gnb0[r, pl.ds(16 * v, 16)] = s * h0v
                return 0
            lax.fori_loop(0, RCH, g0row, 0)
            pltpu.sync_copy(gnb0, g_hbm.at[c].at[pl.ds(rb, RCH), :])
            return 0
        lax.fori_loop(0, NRCH, g0chunk, 0)
        plsc.subcore_barrier()

        def issue(j, buf, sem):
            return pltpu.async_copy(g_hbm.at[c].at[src_idx.at[j]], buf, sem)

        def bread(u, ab, hb, sa, sh):
            rb = pl.multiple_of(base + u * RCH, RCH)
            pltpu.async_copy(acc_sp.at[pl.ds(rb, RCH), :], ab, sa)
            pltpu.async_copy(h0_hbm.at[c].at[pl.ds(rb, RCH), :], hb, sh)

        def bwait(u, ab, hb, sa, sh):
            rb = pl.multiple_of(base + u * RCH, RCH)
            pltpu.make_async_copy(
                acc_sp.at[pl.ds(rb, RCH), :], ab, sa).wait()
            pltpu.make_async_copy(
                h0_hbm.at[c].at[pl.ds(rb, RCH), :], hb, sh).wait()

        def gwrite(u, gb, sw):
            rb = pl.multiple_of(base + u * RCH, RCH)
            return pltpu.async_copy(gb, g_hbm.at[c].at[pl.ds(rb, RCH), :], sw)

        def gwdrain(u, gb, sw):
            rb = pl.multiple_of(base + u * RCH, RCH)
            pltpu.make_async_copy(gb, g_hbm.at[c].at[pl.ds(rb, RCH), :],
                                  sw).wait()

        gbufs = (gbuf0, gbuf1, gbuf2, gbuf3, gbuf4)
        gsems = (sem0, sem1, sem2, sem3, sem4)

        def step(_k, _):
            for bb in range(4):
                issue(bb, gbufs[bb], gsems[bb])

            def chunk(j, _):
                nxt = j + 4
                for b in range(5):
                    @pl.when(jnp.logical_and(nxt < NCHUNK, nxt % 5 == b))
                    def _(b=b):
                        issue(nxt, gbufs[b], gsems[b])

                for b in range(5):
                    @pl.when(j % 5 == b)
                    def _(b=b):
                        pltpu.make_async_copy(
                            g_hbm.at[c].at[src_idx.at[j]],
                            gbufs[b], gsems[b]).wait()
                        pltpu.sync_copy(gbufs[b], acc_sp.at[dst_idx.at[j]],
                                        add=True)
                return 0
            lax.fori_loop(0, NCHUNK, chunk, 0)
            plsc.subcore_barrier()

            bread(0, accb0, h0b0, sra0, srh0)

            def bcompute(u, ab, hb, gb):
                def brow(r, _):
                    li = u * RCH + r
                    s_ns = ns_l[pl.ds(li, 16)][0]
                    c1 = (1.0 - ALPHA) * s_ns * nd_l[pl.ds(li, 16)][0]
                    c2 = ALPHA * s_ns
                    for v in range(4):
                        av = ab[r, pl.ds(16 * v, 16)]
                        hv = hb[r, pl.ds(16 * v, 16)]
                        gb[r, pl.ds(16 * v, 16)] = c1 * av + c2 * hv
                    return 0
                lax.fori_loop(0, RCH, brow, 0)

            def bchunk(u, _):
                nxt = u + 1

                @pl.when(jnp.logical_and(nxt < NRCH, nxt % 2 == 1))
                def _():
                    bread(nxt, accb1, h0b1, sra1, srh1)

                @pl.when(jnp.logical_and(nxt < NRCH, nxt % 2 == 0))
                def _():
                    bread(nxt, accb0, h0b0, sra0, srh0)

                @pl.when(u % 2 == 0)
                def _():
                    bwait(u, accb0, h0b0, sra0, srh0)

                    @pl.when(u >= 2)
                    def _():
                        gwdrain(u - 2, gnb0, swg0)
                    bcompute(u, accb0, h0b0, gnb0)
                    gwrite(u, gnb0, swg0)

                @pl.when(u % 2 == 1)
                def _():
                    bwait(u, accb1, h0b1, sra1, srh1)

                    @pl.when(u >= 2)
                    def _():
                        gwdrain(u - 2, gnb1, swg1)
                    bcompute(u, accb1, h0b1, gnb1)
                    gwrite(u, gnb1, swg1)

                rb = pl.multiple_of(base + u * RCH, RCH)
                pltpu.sync_copy(zbuf, acc_sp.at[pl.ds(rb, RCH), :])
                return 0
            lax.fori_loop(0, NRCH, bchunk, 0)
            gwdrain(NRCH - 2, gnb0, swg0)
            gwdrain(NRCH - 1, gnb1, swg1)
            plsc.subcore_barrier()
            return 0
        lax.fori_loop(0, KSTEPS, step, 0)

        @pl.when(c == 0)
        def _():
            pltpu.sync_copy(inv_l.at[pl.ds(0, DWIN)],
                            inv_hbm.at[pl.ds(base, DWIN)])

    f32 = jnp.float32
    kern = pl.kernel(
        body,
        out_type=[
            jax.ShapeDtypeStruct((NC, NP, FH), f32),
            jax.ShapeDtypeStruct((NP,), f32),
        ],
        mesh=mesh,
        compiler_params=pltpu.CompilerParams(use_tc_tiling_on_sc=False),
        scratch_types=[
            pltpu.VMEM_SHARED((NP, FH), f32),
            pltpu.VMEM_SHARED((NP,), f32),
            pltpu.VMEM_SHARED((NP,), f32),
            pltpu.VMEM((NCHUNK, CH), jnp.int32),
            pltpu.VMEM((NCHUNK, CH), jnp.int32),
            pltpu.VMEM((CH, FH), f32),
            pltpu.VMEM((CH, FH), f32),
            pltpu.VMEM((CH, FH), f32),
            pltpu.VMEM((CH, FH), f32),
            pltpu.VMEM((CH, FH), f32),
            pltpu.VMEM((RCH, FH), f32),
            pltpu.VMEM((RCH, FH), f32),
            pltpu.VMEM((RCH, FH), f32),
            pltpu.VMEM((RCH, FH), f32),
            pltpu.VMEM((RCH, FH), f32),
            pltpu.VMEM((RCH, FH), f32),
            pltpu.VMEM((RCH, FH), f32),
            pltpu.VMEM((RCH,), f32),
            pltpu.VMEM((CH,), f32),
            pltpu.VMEM((DWIN,), f32),
            pltpu.VMEM((LBUF,), f32),
            pltpu.VMEM((LBUF,), f32),
            pltpu.VMEM((LBUF,), f32),
            pltpu.SemaphoreType.DMA,
            pltpu.SemaphoreType.DMA,
            pltpu.SemaphoreType.DMA,
            pltpu.SemaphoreType.DMA,
            pltpu.SemaphoreType.DMA,
            pltpu.SemaphoreType.DMA,
            pltpu.SemaphoreType.DMA,
            pltpu.SemaphoreType.DMA,
            pltpu.SemaphoreType.DMA,
            pltpu.SemaphoreType.DMA,
            pltpu.SemaphoreType.DMA,
        ],
    )
    g, inv_ns = kern(srcp, dstp, h0)
    return g, inv_ns


def _tc_linear(feat, W, b):
    def body(feat_ref, w_ref, b_ref, out_ref):
        h = lax.dot_general(feat_ref[...], w_ref[...],
                            (((1,), (1,)), ((), ())),
                            preferred_element_type=jnp.float32)
        h = jnp.maximum(h + b_ref[...][None, :], 0.0)
        out_ref[0, :N] = h[:, :FH]
        out_ref[1, :N] = h[:, FH:]
        pad = jnp.zeros((NP - N, FH), jnp.float32)
        out_ref[0, N:] = pad
        out_ref[1, N:] = pad

    return pl.pallas_call(
        body,
        out_shape=jax.ShapeDtypeStruct((NC, NP, FH), jnp.float32),
    )(feat, W, b)


def _tc_batchnorm(hfin, inv_ns, gamma, beta):
    def body(h_ref, inv_ref, g_ref, b_ref, out_ref):
        inv = inv_ref[pl.ds(0, N)][:, None]
        for half in range(NC):
            x = h_ref[half, :N] * inv
            m = jnp.mean(x, axis=0)
            var = jnp.mean((x - m[None, :]) ** 2, axis=0)
            scale = lax.rsqrt(var + BN_EPS) * g_ref[pl.ds(half * FH, FH)]
            out_ref[:, pl.ds(half * FH, FH)] = (
                (x - m[None, :]) * scale[None, :]
                + b_ref[pl.ds(half * FH, FH)][None, :])

    return pl.pallas_call(
        body,
        out_shape=jax.ShapeDtypeStruct((N, F), jnp.float32),
    )(hfin, inv_ns, gamma, beta)


@jax.jit
def kernel(feat, edge_index, W, b, gamma, beta):
    src = edge_index[0].astype(jnp.int32)
    dst = edge_index[1].astype(jnp.int32)
    pad = (N + jnp.arange(NT, dtype=jnp.int32))[:, None] * jnp.ones(
        (1, EPAD), jnp.int32)
    srcp = jnp.concatenate([src.reshape(NT, EPT), pad], axis=1)
    srcp = srcp.reshape(NT, NCHUNK, CH)
    dstp = jnp.concatenate([dst.reshape(NT, EPT), pad], axis=1)
    dstp = dstp.reshape(NT, NCHUNK, CH)

    h0 = _tc_linear(feat, W, b)
    g, inv_ns = _sc_propagate(srcp, dstp, h0)
    return _tc_batchnorm(g, inv_ns, gamma, beta)
